# baseline (speedup 1.0000x reference)
import functools

import jax
import jax.numpy as jnp
from jax import lax
from jax.experimental import pallas as pl
from jax.experimental.pallas import tpu as pltpu
from jax.experimental.pallas import tpu_sc as plsc

_TILE = 512
_CPAD = 1024
_BIG = 1e30


def _argmin_body(x_ref, c_ref, cn_ref, o_ref):
    x = x_ref[...]
    prod = jnp.dot(x, c_ref[...], preferred_element_type=jnp.float32)
    dist = cn_ref[...] - 2.0 * prod
    dist = dist + jnp.sum(x * x, axis=1, keepdims=True)
    m = jnp.min(dist, axis=1, keepdims=True)
    col = lax.broadcasted_iota(jnp.int32, dist.shape, 1)
    cand = jnp.where(dist <= m, col, _CPAD)
    o_ref[...] = jnp.min(cand, axis=1)[None, None, :]


def _cluster_ids(x, c_pad, cn_pad):
    n = x.shape[0]
    grid = n // _TILE
    out = pl.pallas_call(
        _argmin_body,
        grid=(grid,),
        in_specs=[
            pl.BlockSpec((_TILE, x.shape[1]), lambda i: (i, 0)),
            pl.BlockSpec((x.shape[1], _CPAD), lambda i: (0, 0)),
            pl.BlockSpec((1, _CPAD), lambda i: (0, 0)),
        ],
        out_specs=pl.BlockSpec((1, 1, _TILE), lambda i: (i, 0, 0)),
        out_shape=jax.ShapeDtypeStruct((grid, 1, _TILE), jnp.int32),
    )(x, c_pad, cn_pad)
    return out.reshape(-1)


def _make_gather(n_rows, d, n_workers, chunk):
    n_chunks = n_rows // (n_workers * chunk)
    mesh = plsc.VectorSubcoreMesh(core_axis_name="c", subcore_axis_name="s")

    @functools.partial(
        pl.kernel,
        mesh=mesh,
        out_type=jax.ShapeDtypeStruct((n_rows, d), jnp.float32),
        scratch_types=[
            pltpu.VMEM((n_chunks, chunk), jnp.int32),
            pltpu.VMEM((chunk, d), jnp.float32),
            pltpu.VMEM((chunk, d), jnp.float32),
            pltpu.SemaphoreType.DMA,
            pltpu.SemaphoreType.DMA,
        ],
    )
    def gather_kernel(table_hbm, idx_hbm, out_hbm, idx_v, buf0, buf1, s0, s1):
        n_cores = mesh.num_cores
        wid = lax.axis_index("s") * n_cores + lax.axis_index("c")
        base = wid * (n_chunks * chunk)
        pltpu.sync_copy(idx_hbm.at[wid], idx_v)
        bufs = (buf0, buf1)
        sems = (s0, s1)
        copies = [None, None]
        for k in range(n_chunks):
            b = k % 2
            if copies[b] is not None:
                copies[b].wait()
                pltpu.sync_copy(
                    bufs[b], out_hbm.at[pl.ds(base + (k - 2) * chunk, chunk)]
                )
            copies[b] = pltpu.async_copy(
                table_hbm.at[idx_v.at[k]], bufs[b], sems[b]
            )
        for k in range(n_chunks - 2, n_chunks):
            b = k % 2
            copies[b].wait()
            pltpu.sync_copy(
                bufs[b], out_hbm.at[pl.ds(base + k * chunk, chunk)]
            )

    return gather_kernel


def kernel(hs, C, Cnorm, emb_table):
    bs, seqlen, size = hs.shape
    n = bs * seqlen
    x = hs.reshape(n, size)
    pad = _CPAD - C.shape[1]
    c_pad = jnp.pad(C, ((0, 0), (0, pad)))
    cn_pad = jnp.pad(Cnorm, ((0, 0), (0, pad)), constant_values=_BIG)

    ids = _cluster_ids(x, c_pad, cn_pad)

    n_workers, chunk = 32, 128
    idx3 = ids.reshape(n_workers, n // (n_workers * chunk), chunk)
    gather = _make_gather(n, size, n_workers, chunk)
    rows = gather(emb_table, idx3)
    return rows.reshape(bs, seqlen // 2, 2 * size)


# speedup vs baseline: 1.0053x; 1.0053x over previous
import functools

import jax
import jax.numpy as jnp
from jax import lax
from jax.experimental import pallas as pl
from jax.experimental.pallas import tpu as pltpu
from jax.experimental.pallas import tpu_sc as plsc

_TILE = 512
_CPAD = 1024
_BIG = 1e30


def _argmin_body(x_ref, c_ref, cn_ref, o_ref):
    x = x_ref[...]
    prod = jnp.dot(x, c_ref[...], preferred_element_type=jnp.float32)
    dist = cn_ref[...] - 2.0 * prod
    dist = dist + jnp.sum(x * x, axis=1, keepdims=True)
    m = jnp.min(dist, axis=1, keepdims=True)
    col = lax.broadcasted_iota(jnp.int32, dist.shape, 1)
    cand = jnp.where(dist <= m, col, _CPAD)
    o_ref[...] = jnp.min(cand, axis=1)[None, None, :]


def _cluster_ids(x, c_pad, cn_pad):
    n = x.shape[0]
    grid = n // _TILE
    out = pl.pallas_call(
        _argmin_body,
        grid=(grid,),
        in_specs=[
            pl.BlockSpec((_TILE, x.shape[1]), lambda i: (i, 0)),
            pl.BlockSpec((x.shape[1], _CPAD), lambda i: (0, 0)),
            pl.BlockSpec((1, _CPAD), lambda i: (0, 0)),
        ],
        out_specs=pl.BlockSpec((1, 1, _TILE), lambda i: (i, 0, 0)),
        out_shape=jax.ShapeDtypeStruct((grid, 1, _TILE), jnp.int32),
    )(x, c_pad, cn_pad)
    return out.reshape(-1)


def _make_gather(n_rows, d, n_workers, chunk):
    n_chunks = n_rows // (n_workers * chunk)
    assert n_chunks == 4
    mesh = plsc.VectorSubcoreMesh(core_axis_name="c", subcore_axis_name="s")

    @functools.partial(
        pl.kernel,
        mesh=mesh,
        out_type=jax.ShapeDtypeStruct((n_rows, d), jnp.float32),
        scratch_types=[
            pltpu.VMEM((chunk,), jnp.int32),
            pltpu.VMEM((chunk,), jnp.int32),
            pltpu.VMEM((chunk,), jnp.int32),
            pltpu.VMEM((chunk,), jnp.int32),
            pltpu.VMEM((chunk, d), jnp.float32),
            pltpu.VMEM((chunk, d), jnp.float32),
            pltpu.VMEM((chunk, d), jnp.float32),
            pltpu.SemaphoreType.DMA,
            pltpu.SemaphoreType.DMA,
            pltpu.SemaphoreType.DMA,
            pltpu.SemaphoreType.DMA,
            pltpu.SemaphoreType.DMA,
        ],
    )
    def gather_kernel(
        table_hbm, idx_hbm, out_hbm,
        i0, i1, i2, i3, b0, b1, b2, g0, g1, g2, w0, w1,
    ):
        wid = lax.axis_index("s") * mesh.num_cores + lax.axis_index("c")
        base = wid * (n_chunks * chunk)
        for k, ik in enumerate((i0, i1, i2, i3)):
            pltpu.sync_copy(idx_hbm.at[wid, k], ik)
        cg0 = pltpu.async_copy(table_hbm.at[i0], b0, g0)
        cg1 = pltpu.async_copy(table_hbm.at[i1], b1, g1)
        cg2 = pltpu.async_copy(table_hbm.at[i2], b2, g2)
        cg0.wait()
        cw0 = pltpu.async_copy(b0, out_hbm.at[pl.ds(base, chunk)], w0)
        cg1.wait()
        cw1 = pltpu.async_copy(b1, out_hbm.at[pl.ds(base + chunk, chunk)], w1)
        cw0.wait()
        cg3 = pltpu.async_copy(table_hbm.at[i3], b0, g0)
        cg2.wait()
        pltpu.sync_copy(b2, out_hbm.at[pl.ds(base + 2 * chunk, chunk)])
        cg3.wait()
        pltpu.sync_copy(b0, out_hbm.at[pl.ds(base + 3 * chunk, chunk)])
        cw1.wait()

    return gather_kernel


def kernel(hs, C, Cnorm, emb_table):
    bs, seqlen, size = hs.shape
    n = bs * seqlen
    x = hs.reshape(n, size)
    pad = _CPAD - C.shape[1]
    c_pad = jnp.pad(C, ((0, 0), (0, pad)))
    cn_pad = jnp.pad(Cnorm, ((0, 0), (0, pad)), constant_values=_BIG)

    ids = _cluster_ids(x, c_pad, cn_pad)

    n_workers, chunk = 32, 128
    idx3 = ids.reshape(n_workers, n // (n_workers * chunk), chunk)
    gather = _make_gather(n, size, n_workers, chunk)
    rows = gather(emb_table, idx3)
    return rows.reshape(bs, seqlen // 2, 2 * size)
